# Initial kernel scaffold; baseline (speedup 1.0000x reference)
import jax
import jax.numpy as jnp
from jax.experimental import pallas as pl


def kernel(image, depth):
    raise NotImplementedError("write your pallas kernel here")


# speedup vs baseline: 47.3573x; 47.3573x over previous
import functools

import jax
import jax.numpy as jnp
from jax import lax
from jax.experimental import pallas as pl
from jax.experimental.pallas import tpu as pltpu
from jax.experimental.pallas import tpu_sc as plsc

B, C, H, W = 8, 3, 512, 512
NC, NS, L = 2, 16, 16
NW = NC * NS
ROWS = B * H
ROWS_PER_W = ROWS // NW
NCHUNK = W // L
_MAGIC = 12582912.0


def _dp_body(image_hbm, depth_hbm, out_l_hbm, out_r_hbm,
             depth_v, img_v, acc_l, acc_r, cnt_l, cnt_r, out_lv, out_rv, sem):
    wid = lax.axis_index("s") * NC + lax.axis_index("c")

    ones = jnp.ones((L,), jnp.float32)
    zeros = jnp.zeros((L,), jnp.float32)
    col0 = lax.iota(jnp.int32, L)

    def zero_body(j, _):
        wb = j * L
        for c in range(C):
            acc_l[pl.ds(c * W + wb, L)] = zeros
            acc_r[pl.ds(c * W + wb, L)] = zeros
        cnt_l[pl.ds(wb, L)] = zeros
        cnt_r[pl.ds(wb, L)] = zeros
        return 0

    lax.fori_loop(0, NCHUNK, zero_body, 0)

    def row_body(i, _):
        r = wid * ROWS_PER_W + i
        b = r // H
        h = r - b * H
        ibase = (b * (C * H) + h) * W

        cps = [pltpu.async_copy(depth_hbm.at[pl.ds(r * W, W)], depth_v, sem)]
        for c in range(C):
            cps.append(pltpu.async_copy(
                image_hbm.at[pl.ds(ibase + c * (H * W), W)],
                img_v.at[pl.ds(c * W, W)], sem))
        for cp in cps:
            cp.wait()

        def scatter_body(k, _):
            wb = k * L
            d = depth_v[pl.ds(wb, L)]
            d = jnp.minimum(jnp.maximum(d, -1024.0), 1024.0)
            s = ((d + _MAGIC) - _MAGIC).astype(jnp.int32)
            colv = col0 + wb
            tl = jnp.clip(colv + s, 0, W - 1)
            tr = jnp.clip(colv - s, 0, W - 1)
            for c in range(C):
                v = img_v[pl.ds(c * W + wb, L)]
                plsc.addupdate_scatter(acc_l, [tl + (c * W)], v)
                plsc.addupdate_scatter(acc_r, [tr + (c * W)], v)
            plsc.addupdate_scatter(cnt_l, [tl], ones)
            plsc.addupdate_scatter(cnt_r, [tr], ones)
            return 0

        lax.fori_loop(0, NCHUNK, scatter_body, 0)

        def norm_body(j, _):
            wb = j * L
            rcl = 1.0 / jnp.maximum(cnt_l[pl.ds(wb, L)], 1.0)
            rcr = 1.0 / jnp.maximum(cnt_r[pl.ds(wb, L)], 1.0)
            for c in range(C):
                out_lv[pl.ds(c * W + wb, L)] = acc_l[pl.ds(c * W + wb, L)] * rcl
                out_rv[pl.ds(c * W + wb, L)] = acc_r[pl.ds(c * W + wb, L)] * rcr
                acc_l[pl.ds(c * W + wb, L)] = zeros
                acc_r[pl.ds(c * W + wb, L)] = zeros
            cnt_l[pl.ds(wb, L)] = zeros
            cnt_r[pl.ds(wb, L)] = zeros
            return 0

        lax.fori_loop(0, NCHUNK, norm_body, 0)

        cps = []
        for c in range(C):
            cps.append(pltpu.async_copy(
                out_lv.at[pl.ds(c * W, W)],
                out_l_hbm.at[pl.ds(ibase + c * (H * W), W)], sem))
            cps.append(pltpu.async_copy(
                out_rv.at[pl.ds(c * W, W)],
                out_r_hbm.at[pl.ds(ibase + c * (H * W), W)], sem))
        for cp in cps:
            cp.wait()
        return 0

    lax.fori_loop(0, ROWS_PER_W, row_body, 0)


@jax.jit
def _dp_merge(img1, dep1):
    mesh = plsc.VectorSubcoreMesh(core_axis_name="c", subcore_axis_name="s")
    f = pl.kernel(
        _dp_body,
        out_type=(
            jax.ShapeDtypeStruct((B * C * H * W,), jnp.float32),
            jax.ShapeDtypeStruct((B * C * H * W,), jnp.float32),
        ),
        mesh=mesh,
        compiler_params=pltpu.CompilerParams(needs_layout_passes=False),
        scratch_types=[
            pltpu.VMEM((W,), jnp.float32),
            pltpu.VMEM((C * W,), jnp.float32),
            pltpu.VMEM((C * W,), jnp.float32),
            pltpu.VMEM((C * W,), jnp.float32),
            pltpu.VMEM((W,), jnp.float32),
            pltpu.VMEM((W,), jnp.float32),
            pltpu.VMEM((C * W,), jnp.float32),
            pltpu.VMEM((C * W,), jnp.float32),
            pltpu.SemaphoreType.DMA,
        ],
    )
    return f(img1, dep1)


def kernel(image, depth):
    img1 = image.reshape(B * C * H * W)
    dep1 = depth.reshape(B * H * W)
    out_l, out_r = _dp_merge(img1, dep1)
    return out_l.reshape(B, C, H, W), out_r.reshape(B, C, H, W)
